# baseline (speedup 1.0000x reference)
import functools

import jax
import jax.numpy as jnp
from jax import lax
from jax.experimental import pallas as pl
from jax.experimental.pallas import tpu as pltpu
from jax.experimental.pallas import tpu_sc as plsc

_IDX_CHUNK = 64


def _lp_body(emb_ref, w_ref, b_ref, lp_ref):
    p = lax.dot_general(
        emb_ref[...], w_ref[...],
        (((1,), (1,)), ((), ())),
        preferred_element_type=jnp.float32,
    )
    p = p + b_ref[...]
    m = jnp.max(p, axis=1, keepdims=True)
    s = jnp.sum(jnp.exp(p - m), axis=1, keepdims=True)
    lp_ref[...] = p - (m + jnp.log(s))


def _make_sc_gather(vocab, batch):
    info = plsc.get_sparse_core_info()
    nc, ns = info.num_cores, info.num_subcores
    nw = nc * ns
    b_per_w = batch // nw
    n_chunks = b_per_w // _IDX_CHUNK
    mesh = plsc.VectorSubcoreMesh(core_axis_name="c", subcore_axis_name="s")

    @functools.partial(
        pl.kernel,
        mesh=mesh,
        out_type=jax.ShapeDtypeStruct((batch, vocab), jnp.float32),
        scratch_types=[
            pltpu.VMEM((b_per_w,), jnp.int32),
            pltpu.VMEM((_IDX_CHUNK, vocab), jnp.float32),
            pltpu.SemaphoreType.DMA,
        ],
        compiler_params=pltpu.CompilerParams(use_tc_tiling_on_sc=False),
    )
    def gather_kernel(lp_hbm, idx_hbm, out_hbm, idx_v, rows_v, sem):
        wid = lax.axis_index("s") * nc + lax.axis_index("c")
        base = wid * b_per_w
        pltpu.sync_copy(idx_hbm.at[pl.ds(base, b_per_w)], idx_v)
        for c in range(n_chunks):
            off = c * _IDX_CHUNK
            pltpu.async_copy(
                lp_hbm.at[idx_v.at[pl.ds(off, _IDX_CHUNK)]],
                rows_v,
                sem,
            ).wait()
            pltpu.sync_copy(rows_v, out_hbm.at[pl.ds(base + off, _IDX_CHUNK)])

    return gather_kernel


def kernel(target_idxs, emb_table, W, b):
    vocab, dim = W.shape
    batch = target_idxs.shape[0]

    lp = pl.pallas_call(
        _lp_body,
        out_shape=jax.ShapeDtypeStruct((vocab, vocab), jnp.float32),
    )(emb_table, W, b.reshape(1, vocab))

    gather = _make_sc_gather(vocab, batch)
    return gather(lp, target_idxs.astype(jnp.int32))
